# Initial kernel scaffold; baseline (speedup 1.0000x reference)
import jax
import jax.numpy as jnp
from jax.experimental import pallas as pl


def kernel(x, adj, W1, b1, g1, be1, W2, b2, g3, be3):
    raise NotImplementedError("write your pallas kernel here")


# speedup vs baseline: 4.9654x; 4.9654x over previous
import functools

import jax
import jax.numpy as jnp
from jax import lax
from jax.experimental import pallas as pl
from jax.experimental.pallas import tpu as pltpu
from jax.experimental.pallas import tpu_sc as plsc

_N = 10000
_E = 320000
_D = 128
_L = 5
_BN_EPS = 1e-5
_NC = 2
_NS = 16
_DH = _D // _NC
_EB = 128
_NBUF = 4
_LEAD = 2
_BLK_PER_TILE = 160
_EPAD = _BLK_PER_TILE * _EB * _NS
_NP = 10240
_ROWS_PER_TILE = _NP // _NS


def _sc_aggregate(xc, srcb, dstb, zeros):
    mesh = plsc.VectorSubcoreMesh(core_axis_name="c", subcore_axis_name="s")

    @functools.partial(
        pl.kernel,
        out_type=jax.ShapeDtypeStruct((_NC, _NP, _DH), jnp.float32),
        mesh=mesh,
        scratch_types=[
            pltpu.VMEM((_BLK_PER_TILE, _EB), jnp.int32),
            pltpu.VMEM((_BLK_PER_TILE, _EB), jnp.int32),
            pltpu.VMEM((_NBUF * _EB, _DH), jnp.float32),
            pltpu.VMEM_SHARED((_NP, _DH), jnp.float32),
            pltpu.SemaphoreType.DMA((_NBUF,)),
        ],
        compiler_params=pltpu.CompilerParams(use_tc_tiling_on_sc=False),
    )
    def body(xc_hbm, srcb_hbm, dstb_hbm, zeros_hbm, out_hbm,
             srcb_v, dstb_v, rows, accum, gsem):
        c = lax.axis_index("c")
        s = lax.axis_index("s")
        r0 = s * _ROWS_PER_TILE
        pltpu.sync_copy(zeros_hbm.at[pl.ds(r0, _ROWS_PER_TILE)],
                        accum.at[pl.ds(r0, _ROWS_PER_TILE)])
        b0 = s * _BLK_PER_TILE
        pltpu.sync_copy(srcb_hbm.at[pl.ds(b0, _BLK_PER_TILE)], srcb_v)
        pltpu.sync_copy(dstb_hbm.at[pl.ds(b0, _BLK_PER_TILE)], dstb_v)
        plsc.subcore_barrier()

        xh = xc_hbm.at[c]

        def g_start(j):
            bb = lax.rem(j, _NBUF)
            pltpu.async_copy(xh.at[srcb_v.at[j]],
                             rows.at[pl.ds(bb * _EB, _EB)], gsem.at[bb])

        def step(t, carry):
            @pl.when(t < _BLK_PER_TILE)
            def _():
                g_start(t)

            @pl.when(t >= _LEAD)
            def _():
                j = t - _LEAD
                bb = lax.rem(j, _NBUF)
                slot = rows.at[pl.ds(bb * _EB, _EB)]
                pltpu.make_async_copy(xh.at[srcb_v.at[j]], slot,
                                      gsem.at[bb]).wait()
                pltpu.sync_copy(slot, accum.at[dstb_v.at[j]], add=True)

            return carry

        lax.fori_loop(0, _BLK_PER_TILE + _LEAD, step, 0)
        plsc.subcore_barrier()
        pltpu.sync_copy(accum.at[pl.ds(r0, _ROWS_PER_TILE)],
                        out_hbm.at[c, pl.ds(r0, _ROWS_PER_TILE)])

    return body(xc, srcb, dstb, zeros)


def _tc_mlp(x, parts, a1, c1, a2, c2):
    nb = 10
    bn = _N // nb

    def body(x_ref, p_ref, a1_ref, c1_ref, a2_ref, c2_ref, out_ref):
        aggr = jnp.concatenate([p_ref[0], p_ref[1]], axis=1)
        h0 = x_ref[...] + aggr
        for l in range(_L):
            t = jnp.dot(h0, a1_ref[l], preferred_element_type=jnp.float32)
            t = jnp.maximum(t + c1_ref[l], 0.0)
            h = jnp.dot(t, a2_ref[l], preferred_element_type=jnp.float32)
            h = h + c2_ref[l]
            if l < _L - 1:
                h = jnp.maximum(h, 0.0)
            out_ref[l] = h

    return pl.pallas_call(
        body,
        grid=(nb,),
        in_specs=[
            pl.BlockSpec((bn, _D), lambda i: (i, 0)),
            pl.BlockSpec((_NC, bn, _DH), lambda i: (0, i, 0)),
            pl.BlockSpec((_L, _D, _D), lambda i: (0, 0, 0)),
            pl.BlockSpec((_L, _D), lambda i: (0, 0)),
            pl.BlockSpec((_L, _D, _D), lambda i: (0, 0, 0)),
            pl.BlockSpec((_L, _D), lambda i: (0, 0)),
        ],
        out_specs=pl.BlockSpec((_L, bn, _D), lambda i: (0, i, 0)),
        out_shape=jax.ShapeDtypeStruct((_L, _N, _D), jnp.float32),
    )(x, parts, a1, c1, a2, c2)


def kernel(x, adj, W1, b1, g1, be1, W2, b2, g3, be3):
    npad = _EPAD - _E
    srcb = jnp.concatenate(
        [adj[0], jnp.zeros((npad,), jnp.int32)]).reshape(-1, _EB)
    dstb = jnp.concatenate(
        [adj[1], jnp.full((npad,), _N, jnp.int32)]).reshape(-1, _EB)
    xc = jnp.stack([x[:, :_DH], x[:, _DH:]])
    zeros = jnp.zeros((_NP, _DH), jnp.float32)
    parts = _sc_aggregate(xc, srcb, dstb, zeros)
    s1 = g1 / jnp.sqrt(1.0 + _BN_EPS)
    s3 = g3 / jnp.sqrt(1.0 + _BN_EPS)
    a1 = jnp.transpose(W1, (0, 2, 1)) * s1[:, None, :]
    c1 = b1 * s1 + be1
    a2 = jnp.transpose(W2, (0, 2, 1)) * s3[:, None, :]
    c2 = b2 * s3 + be3
    return _tc_mlp(x, parts, a1, c1, a2, c2)


# speedup vs baseline: 6.0417x; 1.2168x over previous
import functools

import jax
import jax.numpy as jnp
from jax import lax
from jax.experimental import pallas as pl
from jax.experimental.pallas import tpu as pltpu
from jax.experimental.pallas import tpu_sc as plsc

_N = 10000
_E = 320000
_D = 128
_L = 5
_BN_EPS = 1e-5
_NC = 2
_NS = 16
_DH = _D // _NC
_EB = 128
_NBUF = 4
_LEAD = 2
_NG = 3
_BLK_PER_TILE = 160
_NBLK = _BLK_PER_TILE * _NS
_EPAD = _NBLK * _EB
_NP = 10240
_NPG = _NG * _NP
_ROWS_PER_TILE = _NPG // _NS


def _sc_aggregate(xcb, srcb, dstb, zeros):
    mesh = plsc.VectorSubcoreMesh(core_axis_name="c", subcore_axis_name="s")

    @functools.partial(
        pl.kernel,
        out_type=jax.ShapeDtypeStruct((_NC, _NPG, _DH), jnp.bfloat16),
        mesh=mesh,
        scratch_types=[
            pltpu.VMEM((_BLK_PER_TILE, _EB), jnp.int32),
            pltpu.VMEM((_BLK_PER_TILE, _EB), jnp.int32),
            pltpu.VMEM((_NBUF * _EB, _DH), jnp.bfloat16),
            pltpu.VMEM_SHARED((_NPG, _DH), jnp.bfloat16),
            pltpu.SemaphoreType.DMA((_NBUF,)),
        ],
        compiler_params=pltpu.CompilerParams(use_tc_tiling_on_sc=False),
    )
    def body(xcb_hbm, srcb_hbm, dstb_hbm, zeros_hbm, out_hbm,
             srcb_v, dstb_v, rows, accum, gsem):
        c = lax.axis_index("c")
        s = lax.axis_index("s")
        r0 = s * _ROWS_PER_TILE
        pltpu.sync_copy(zeros_hbm.at[pl.ds(r0, _ROWS_PER_TILE)],
                        accum.at[pl.ds(r0, _ROWS_PER_TILE)])
        b0 = s * _BLK_PER_TILE
        pltpu.sync_copy(srcb_hbm.at[pl.ds(b0, _BLK_PER_TILE)], srcb_v)
        pltpu.sync_copy(dstb_hbm.at[pl.ds(b0, _BLK_PER_TILE)], dstb_v)
        plsc.subcore_barrier()

        xh = xcb_hbm.at[c]

        def g_start(j):
            bb = lax.rem(j, _NBUF)
            pltpu.async_copy(xh.at[srcb_v.at[j]],
                             rows.at[pl.ds(bb * _EB, _EB)], gsem.at[bb])

        def step(t, carry):
            @pl.when(t < _BLK_PER_TILE)
            def _():
                g_start(t)

            @pl.when(t >= _LEAD)
            def _():
                j = t - _LEAD
                bb = lax.rem(j, _NBUF)
                slot = rows.at[pl.ds(bb * _EB, _EB)]
                pltpu.make_async_copy(xh.at[srcb_v.at[j]], slot,
                                      gsem.at[bb]).wait()
                pltpu.sync_copy(slot, accum.at[dstb_v.at[j]], add=True)

            return carry

        lax.fori_loop(0, _BLK_PER_TILE + _LEAD, step, 0)
        plsc.subcore_barrier()
        pltpu.sync_copy(accum.at[pl.ds(r0, _ROWS_PER_TILE)],
                        out_hbm.at[c, pl.ds(r0, _ROWS_PER_TILE)])

    return body(xcb, srcb, dstb, zeros)


def _tc_mlp(x, parts, a1, c1, a2, c2):
    nb = 10
    bn = _N // nb

    def body(x_ref, p_ref, a1_ref, c1_ref, a2_ref, c2_ref, out_ref):
        halves = []
        for ci in range(_NC):
            acc = p_ref[ci, 0].astype(jnp.float32)
            for g in range(1, _NG):
                acc = acc + p_ref[ci, g].astype(jnp.float32)
            halves.append(acc)
        h0 = x_ref[...] + jnp.concatenate(halves, axis=1)
        for l in range(_L):
            t = jnp.dot(h0, a1_ref[l], preferred_element_type=jnp.float32)
            t = jnp.maximum(t + c1_ref[l], 0.0)
            h = jnp.dot(t, a2_ref[l], preferred_element_type=jnp.float32)
            h = h + c2_ref[l]
            if l < _L - 1:
                h = jnp.maximum(h, 0.0)
            out_ref[l] = h

    return pl.pallas_call(
        body,
        grid=(nb,),
        in_specs=[
            pl.BlockSpec((bn, _D), lambda i: (i, 0)),
            pl.BlockSpec((_NC, _NG, bn, _DH), lambda i: (0, 0, i, 0)),
            pl.BlockSpec((_L, _D, _D), lambda i: (0, 0, 0)),
            pl.BlockSpec((_L, _D), lambda i: (0, 0)),
            pl.BlockSpec((_L, _D, _D), lambda i: (0, 0, 0)),
            pl.BlockSpec((_L, _D), lambda i: (0, 0)),
        ],
        out_specs=pl.BlockSpec((_L, bn, _D), lambda i: (0, i, 0)),
        out_shape=jax.ShapeDtypeStruct((_L, _N, _D), jnp.float32),
    )(x, parts, a1, c1, a2, c2)


def kernel(x, adj, W1, b1, g1, be1, W2, b2, g3, be3):
    npad = _EPAD - _E
    srcb = jnp.concatenate(
        [adj[0], jnp.zeros((npad,), jnp.int32)]).reshape(_NBLK, _EB)
    dstb = jnp.concatenate(
        [adj[1], jnp.full((npad,), _N, jnp.int32)]).reshape(_NBLK, _EB)
    goff = (jnp.arange(_NBLK, dtype=jnp.int32) % _NG) * _NP
    dstb = dstb + goff[:, None]
    xcb = jnp.stack([x[:, :_DH], x[:, _DH:]]).astype(jnp.bfloat16)
    zeros = jnp.zeros((_NPG, _DH), jnp.bfloat16)
    parts = _sc_aggregate(xcb, srcb, dstb, zeros)
    parts = parts.reshape(_NC, _NG, _NP, _DH)
    s1 = g1 / jnp.sqrt(1.0 + _BN_EPS)
    s3 = g3 / jnp.sqrt(1.0 + _BN_EPS)
    a1 = jnp.transpose(W1, (0, 2, 1)) * s1[:, None, :]
    c1 = b1 * s1 + be1
    a2 = jnp.transpose(W2, (0, 2, 1)) * s3[:, None, :]
    c2 = b2 * s3 + be3
    return _tc_mlp(x, parts, a1, c1, a2, c2)


# speedup vs baseline: 6.1790x; 1.0227x over previous
import functools

import jax
import jax.numpy as jnp
from jax import lax
from jax.experimental import pallas as pl
from jax.experimental.pallas import tpu as pltpu
from jax.experimental.pallas import tpu_sc as plsc

_N = 10000
_E = 320000
_D = 128
_L = 5
_BN_EPS = 1e-5
_NC = 2
_NS = 16
_DH = _D // _NC
_EB = 128
_NBUF = 4
_LEAD = 3
_NG = 3
_BLK_PER_TILE = 160
_NBLK = _BLK_PER_TILE * _NS
_EPAD = _NBLK * _EB
_NP = 10240
_NPG = _NG * _NP
_ROWS_PER_TILE = _NPG // _NS


def _sc_aggregate(xcb, srcb, dstb, zeros):
    mesh = plsc.VectorSubcoreMesh(core_axis_name="c", subcore_axis_name="s")

    @functools.partial(
        pl.kernel,
        out_type=jax.ShapeDtypeStruct((_NC, _NPG, _DH), jnp.bfloat16),
        mesh=mesh,
        scratch_types=[
            pltpu.VMEM((_BLK_PER_TILE, _EB), jnp.int32),
            pltpu.VMEM((_BLK_PER_TILE, _EB), jnp.int32),
            pltpu.VMEM((_NBUF * _EB, _DH), jnp.bfloat16),
            pltpu.VMEM_SHARED((_NPG, _DH), jnp.bfloat16),
            pltpu.SemaphoreType.DMA((_NBUF,)),
        ],
        compiler_params=pltpu.CompilerParams(use_tc_tiling_on_sc=False),
    )
    def body(xcb_hbm, srcb_hbm, dstb_hbm, zeros_hbm, out_hbm,
             srcb_v, dstb_v, rows, accum, gsem):
        c = lax.axis_index("c")
        s = lax.axis_index("s")
        r0 = s * _ROWS_PER_TILE
        pltpu.sync_copy(zeros_hbm.at[pl.ds(r0, _ROWS_PER_TILE)],
                        accum.at[pl.ds(r0, _ROWS_PER_TILE)])
        b0 = s * _BLK_PER_TILE
        pltpu.sync_copy(srcb_hbm.at[pl.ds(b0, _BLK_PER_TILE)], srcb_v)
        pltpu.sync_copy(dstb_hbm.at[pl.ds(b0, _BLK_PER_TILE)], dstb_v)
        plsc.subcore_barrier()

        xh = xcb_hbm.at[c]

        def g_start(j):
            bb = lax.rem(j, _NBUF)
            pltpu.async_copy(xh.at[srcb_v.at[j]],
                             rows.at[pl.ds(bb * _EB, _EB)], gsem.at[bb])

        def step(t, carry):
            @pl.when(t < _BLK_PER_TILE)
            def _():
                g_start(t)

            @pl.when(t >= _LEAD)
            def _():
                j = t - _LEAD
                bb = lax.rem(j, _NBUF)
                slot = rows.at[pl.ds(bb * _EB, _EB)]
                pltpu.make_async_copy(xh.at[srcb_v.at[j]], slot,
                                      gsem.at[bb]).wait()
                pltpu.sync_copy(slot, accum.at[dstb_v.at[j]], add=True)

            return carry

        lax.fori_loop(0, _BLK_PER_TILE + _LEAD, step, 0)
        plsc.subcore_barrier()
        pltpu.sync_copy(accum.at[pl.ds(r0, _ROWS_PER_TILE)],
                        out_hbm.at[c, pl.ds(r0, _ROWS_PER_TILE)])

    return body(xcb, srcb, dstb, zeros)


def _tc_mlp(x, parts, a1, c1, a2, c2):
    nb = 10
    bn = _N // nb

    def body(x_ref, p_ref, a1_ref, c1_ref, a2_ref, c2_ref, out_ref):
        halves = []
        for ci in range(_NC):
            acc = p_ref[ci, 0].astype(jnp.float32)
            for g in range(1, _NG):
                acc = acc + p_ref[ci, g].astype(jnp.float32)
            halves.append(acc)
        h0 = x_ref[...] + jnp.concatenate(halves, axis=1)
        for l in range(_L):
            t = jnp.dot(h0, a1_ref[l], preferred_element_type=jnp.float32)
            t = jnp.maximum(t + c1_ref[l], 0.0)
            h = jnp.dot(t, a2_ref[l], preferred_element_type=jnp.float32)
            h = h + c2_ref[l]
            if l < _L - 1:
                h = jnp.maximum(h, 0.0)
            out_ref[l] = h

    return pl.pallas_call(
        body,
        grid=(nb,),
        in_specs=[
            pl.BlockSpec((bn, _D), lambda i: (i, 0)),
            pl.BlockSpec((_NC, _NG, bn, _DH), lambda i: (0, 0, i, 0)),
            pl.BlockSpec((_L, _D, _D), lambda i: (0, 0, 0)),
            pl.BlockSpec((_L, _D), lambda i: (0, 0)),
            pl.BlockSpec((_L, _D, _D), lambda i: (0, 0, 0)),
            pl.BlockSpec((_L, _D), lambda i: (0, 0)),
        ],
        out_specs=pl.BlockSpec((_L, bn, _D), lambda i: (0, i, 0)),
        out_shape=jax.ShapeDtypeStruct((_L, _N, _D), jnp.float32),
    )(x, parts, a1, c1, a2, c2)


def kernel(x, adj, W1, b1, g1, be1, W2, b2, g3, be3):
    npad = _EPAD - _E
    srcb = jnp.concatenate(
        [adj[0], jnp.zeros((npad,), jnp.int32)]).reshape(_NBLK, _EB)
    dstb = jnp.concatenate(
        [adj[1], jnp.full((npad,), _N, jnp.int32)]).reshape(_NBLK, _EB)
    goff = (jnp.arange(_NBLK, dtype=jnp.int32) % _NG) * _NP
    dstb = dstb + goff[:, None]
    xcb = jnp.stack([x[:, :_DH], x[:, _DH:]]).astype(jnp.bfloat16)
    zeros = jnp.zeros((_NPG, _DH), jnp.bfloat16)
    parts = _sc_aggregate(xcb, srcb, dstb, zeros)
    parts = parts.reshape(_NC, _NG, _NP, _DH)
    s1 = g1 / jnp.sqrt(1.0 + _BN_EPS)
    s3 = g3 / jnp.sqrt(1.0 + _BN_EPS)
    a1 = jnp.transpose(W1, (0, 2, 1)) * s1[:, None, :]
    c1 = b1 * s1 + be1
    a2 = jnp.transpose(W2, (0, 2, 1)) * s3[:, None, :]
    c2 = b2 * s3 + be3
    return _tc_mlp(x, parts, a1, c1, a2, c2)


# speedup vs baseline: 6.2015x; 1.0036x over previous
import functools

import jax
import jax.numpy as jnp
from jax import lax
from jax.experimental import pallas as pl
from jax.experimental.pallas import tpu as pltpu
from jax.experimental.pallas import tpu_sc as plsc

_N = 10000
_E = 320000
_D = 128
_L = 5
_BN_EPS = 1e-5
_NC = 2
_NS = 16
_DH = _D // _NC
_EB = 128
_NG = 3
_BLK_PER_TILE = 160
_PAIRS = _BLK_PER_TILE // 2
_NBLK = _BLK_PER_TILE * _NS
_EPAD = _NBLK * _EB
_NP = 10112
_NPG = _NG * _NP
_ROWS_PER_TILE = _NPG // _NS


def _sc_aggregate(xcb, srcb, dstb, zeros):
    mesh = plsc.VectorSubcoreMesh(core_axis_name="c", subcore_axis_name="s")

    @functools.partial(
        pl.kernel,
        out_type=jax.ShapeDtypeStruct((_NC, _NPG, _DH), jnp.bfloat16),
        mesh=mesh,
        scratch_types=[
            pltpu.VMEM((_BLK_PER_TILE, _EB), jnp.int32),
            pltpu.VMEM((_BLK_PER_TILE, _EB), jnp.int32),
            pltpu.VMEM((2 * _EB, _DH), jnp.bfloat16),
            pltpu.VMEM((2 * _EB, _DH), jnp.bfloat16),
            pltpu.VMEM_SHARED((_NPG, _DH), jnp.bfloat16),
            pltpu.SemaphoreType.DMA((2,)),
            pltpu.SemaphoreType.DMA((2,)),
        ],
        compiler_params=pltpu.CompilerParams(use_tc_tiling_on_sc=False),
    )
    def body(xcb_hbm, srcb_hbm, dstb_hbm, zeros_hbm, out_hbm,
             srcb_v, dstb_v, rows_a, rows_b, accum, gsem_a, gsem_b):
        c = lax.axis_index("c")
        s = lax.axis_index("s")
        r0 = s * _ROWS_PER_TILE
        pltpu.sync_copy(zeros_hbm.at[pl.ds(r0, _ROWS_PER_TILE)],
                        accum.at[pl.ds(r0, _ROWS_PER_TILE)])
        b0 = s * _BLK_PER_TILE
        pltpu.sync_copy(srcb_hbm.at[pl.ds(b0, _BLK_PER_TILE)], srcb_v)
        pltpu.sync_copy(dstb_hbm.at[pl.ds(b0, _BLK_PER_TILE)], dstb_v)
        plsc.subcore_barrier()

        xh = xcb_hbm.at[c]

        def step(t, carry):
            @pl.when(t < _PAIRS)
            def _():
                bb = lax.rem(t, 2)
                pltpu.async_copy(xh.at[srcb_v.at[2 * t]],
                                 rows_a.at[pl.ds(bb * _EB, _EB)],
                                 gsem_a.at[bb])
                pltpu.async_copy(xh.at[srcb_v.at[2 * t + 1]],
                                 rows_b.at[pl.ds(bb * _EB, _EB)],
                                 gsem_b.at[bb])

            @pl.when(t >= 2)
            def _():
                k = t - 2
                bb = lax.rem(k, 2)
                slot_a = rows_a.at[pl.ds(bb * _EB, _EB)]
                pltpu.make_async_copy(xh.at[srcb_v.at[2 * k]], slot_a,
                                      gsem_a.at[bb]).wait()
                pltpu.sync_copy(slot_a, accum.at[dstb_v.at[2 * k]], add=True)
                slot_b = rows_b.at[pl.ds(bb * _EB, _EB)]
                pltpu.make_async_copy(xh.at[srcb_v.at[2 * k + 1]], slot_b,
                                      gsem_b.at[bb]).wait()
                pltpu.sync_copy(slot_b, accum.at[dstb_v.at[2 * k + 1]],
                                add=True)

            return carry

        lax.fori_loop(0, _PAIRS + 2, step, 0)
        plsc.subcore_barrier()
        pltpu.sync_copy(accum.at[pl.ds(r0, _ROWS_PER_TILE)],
                        out_hbm.at[c, pl.ds(r0, _ROWS_PER_TILE)])

    return body(xcb, srcb, dstb, zeros)


def _tc_mlp(x, parts, a1, c1, a2, c2):
    nb = 10
    bn = _N // nb

    def body(x_ref, p_ref, a1_ref, c1_ref, a2_ref, c2_ref, out_ref):
        halves = []
        for ci in range(_NC):
            acc = p_ref[ci, 0].astype(jnp.float32)
            for g in range(1, _NG):
                acc = acc + p_ref[ci, g].astype(jnp.float32)
            halves.append(acc)
        h0 = x_ref[...] + jnp.concatenate(halves, axis=1)
        for l in range(_L):
            t = jnp.dot(h0, a1_ref[l], preferred_element_type=jnp.float32)
            t = jnp.maximum(t + c1_ref[l], 0.0)
            h = jnp.dot(t, a2_ref[l], preferred_element_type=jnp.float32)
            h = h + c2_ref[l]
            if l < _L - 1:
                h = jnp.maximum(h, 0.0)
            out_ref[l] = h

    return pl.pallas_call(
        body,
        grid=(nb,),
        in_specs=[
            pl.BlockSpec((bn, _D), lambda i: (i, 0)),
            pl.BlockSpec((_NC, _NG, bn, _DH), lambda i: (0, 0, i, 0)),
            pl.BlockSpec((_L, _D, _D), lambda i: (0, 0, 0)),
            pl.BlockSpec((_L, _D), lambda i: (0, 0)),
            pl.BlockSpec((_L, _D, _D), lambda i: (0, 0, 0)),
            pl.BlockSpec((_L, _D), lambda i: (0, 0)),
        ],
        out_specs=pl.BlockSpec((_L, bn, _D), lambda i: (0, i, 0)),
        out_shape=jax.ShapeDtypeStruct((_L, _N, _D), jnp.float32),
    )(x, parts, a1, c1, a2, c2)


def kernel(x, adj, W1, b1, g1, be1, W2, b2, g3, be3):
    npad = _EPAD - _E
    srcb = jnp.concatenate(
        [adj[0], jnp.zeros((npad,), jnp.int32)]).reshape(_NBLK, _EB)
    dstb = jnp.concatenate(
        [adj[1], jnp.full((npad,), _N, jnp.int32)]).reshape(_NBLK, _EB)
    goff = (jnp.arange(_NBLK, dtype=jnp.int32) % _NG) * _NP
    dstb = dstb + goff[:, None]
    xcb = jnp.stack([x[:, :_DH], x[:, _DH:]]).astype(jnp.bfloat16)
    zeros = jnp.zeros((_NPG, _DH), jnp.bfloat16)
    parts = _sc_aggregate(xcb, srcb, dstb, zeros)
    parts = parts.reshape(_NC, _NG, _NP, _DH)
    s1 = g1 / jnp.sqrt(1.0 + _BN_EPS)
    s3 = g3 / jnp.sqrt(1.0 + _BN_EPS)
    a1 = jnp.transpose(W1, (0, 2, 1)) * s1[:, None, :]
    c1 = b1 * s1 + be1
    a2 = jnp.transpose(W2, (0, 2, 1)) * s3[:, None, :]
    c2 = b2 * s3 + be3
    return _tc_mlp(x, parts, a1, c1, a2, c2)
